# baseline (speedup 1.0000x reference)
import functools

import jax
import jax.numpy as jnp
from jax import lax
from jax.experimental import pallas as pl
from jax.experimental.pallas import tpu as pltpu
from jax.experimental.pallas import tpu_sc as plsc

_B = 16
_L = 4096
_D = 512
_TOTAL = _B * _L // 2
_NW = 32
_RPW = _B * _L // _NW
_CHUNK = 64
_NCHUNK = _B * _L // _CHUNK
_CPW = _NCHUNK // _NW
_CPS = _L // _CHUNK
_LANES = 16


def _sc_body(flat_hbm, starts_hbm, ends_hbm, out_hbm, mask_hbm,
             st_v, en_v, idx0, idx1, buf0, buf1, zbuf, mbuf,
             isem0, isem1, osem0, osem1, zsem):
    cid = lax.axis_index("c")
    sid = lax.axis_index("s")
    w = sid * 2 + cid

    pltpu.sync_copy(starts_hbm, st_v.at[pl.ds(0, _LANES)])
    pltpu.sync_copy(ends_hbm, en_v.at[pl.ds(0, _LANES)])
    iota = lax.iota(jnp.int32, _LANES)
    zerosf = jnp.zeros((_LANES,), jnp.float32)

    bufs = (buf0, buf1)
    idxs = (idx0, idx1)
    isems = (isem0, isem1)
    osems = (osem0, osem1)

    def _zrow(row, _):
        for kk in range(_D // _LANES):
            zbuf[row, pl.ds(kk * _LANES, _LANES)] = zerosf
        return 0
    lax.fori_loop(0, _CHUNK, _zrow, 0)

    mb = w // 2
    mt0 = (w % 2) * _RPW
    mstart = st_v[pl.ds(mb, _LANES)][0]
    mend = en_v[pl.ds(mb, _LANES)][0]
    mnv = jnp.clip(mend - mstart - mt0, 0, _RPW)
    for j in range(_RPW // _LANES):
        m = jnp.where(j * _LANES + iota < mnv, 1.0, 0.0).astype(jnp.float32)
        mbuf[pl.ds(j * _LANES, _LANES)] = m
    pltpu.sync_copy(
        mbuf, mask_hbm.at[pl.ds(pl.multiple_of(w * _RPW, _RPW), _RPW)])

    def chunk_info(i):
        c_glob = i * _NW + w
        bseg = c_glob // _CPS
        trow = (c_glob % _CPS) * _CHUNK
        st = st_v[pl.ds(bseg, _LANES)][0]
        en = en_v[pl.ds(bseg, _LANES)][0]
        nvc = jnp.clip(en - st - trow, 0, _CHUNK)
        return nvc, st + trow, pl.multiple_of(c_glob * _CHUNK, _CHUNK)

    def start_gather(s, bi):
        for kk in range(_CHUNK // _LANES):
            v = jnp.minimum(s + kk * _LANES + iota, _TOTAL - 1)
            idxs[bi][pl.ds(kk * _LANES, _LANES)] = v
        pltpu.make_async_copy(
            flat_hbm.at[idxs[bi]], bufs[bi], isems[bi]).start()

    for bi in range(2):
        nvc_p, s_p, _ = chunk_info(bi)

        @pl.when(nvc_p > 0)
        def _(s_p=s_p, bi=bi):
            start_gather(s_p, bi)

    def _slot(i, bi, pend_bi, nz):
        nvc, s, dst = chunk_info(i)

        @pl.when(nvc > 0)
        def _():
            pltpu.make_async_copy(
                flat_hbm.at[idxs[bi]], bufs[bi], isems[bi]).wait()

            def _ztail(row, _c):
                for kk in range(_D // _LANES):
                    bufs[bi][row, pl.ds(kk * _LANES, _LANES)] = zerosf
                return 0
            lax.fori_loop(nvc, _CHUNK, _ztail, 0)

            pltpu.make_async_copy(
                bufs[bi], out_hbm.at[pl.ds(dst, _CHUNK)], osems[bi]).start()

        @pl.when(nvc == 0)
        def _():
            pltpu.make_async_copy(
                zbuf, out_hbm.at[pl.ds(dst, _CHUNK)], zsem).start()

        pend = jnp.where(nvc > 0, jnp.int32(1), pend_bi)

        if i + 2 < _CPW:
            nvc_n, s_n, _ = chunk_info(i + 2)

            @pl.when(nvc_n > 0)
            def _():
                @pl.when(pend > 0)
                def _():
                    pltpu.make_async_copy(
                        bufs[bi], out_hbm.at[pl.ds(0, _CHUNK)], osems[bi]
                    ).wait()
                start_gather(s_n, bi)

            pend = jnp.where(nvc_n > 0, jnp.int32(0), pend)

        nz_new = nz + jnp.where(nvc == 0, jnp.int32(1), jnp.int32(0))
        return pend, nz_new

    u0 = jnp.int32(0)
    u1 = jnp.int32(0)
    nz = jnp.int32(0)
    for i in range(_CPW):
        if i % 2 == 0:
            u0, nz = _slot(i, 0, u0, nz)
        else:
            u1, nz = _slot(i, 1, u1, nz)

    for bi, u in ((0, u0), (1, u1)):
        @pl.when(u > 0)
        def _(bi=bi):
            pltpu.make_async_copy(
                bufs[bi], out_hbm.at[pl.ds(0, _CHUNK)], osems[bi]
            ).wait()

    def _zdrain(h, _):
        pltpu.make_async_copy(
            zbuf, out_hbm.at[pl.ds(0, _CHUNK)], zsem).wait()
        return 0
    lax.fori_loop(0, nz, _zdrain, 0)


@jax.jit
def _padded_gather(flat, starts, ends):
    mesh = plsc.VectorSubcoreMesh(core_axis_name="c", subcore_axis_name="s")
    return pl.kernel(
        _sc_body,
        out_type=(
            jax.ShapeDtypeStruct((_B * _L, _D), jnp.float32),
            jax.ShapeDtypeStruct((_B * _L,), jnp.float32),
        ),
        mesh=mesh,
        scratch_types=[
            pltpu.VMEM((2 * _LANES,), jnp.int32),
            pltpu.VMEM((2 * _LANES,), jnp.int32),
            pltpu.VMEM((_CHUNK,), jnp.int32),
            pltpu.VMEM((_CHUNK,), jnp.int32),
            pltpu.VMEM((_CHUNK, _D), jnp.float32),
            pltpu.VMEM((_CHUNK, _D), jnp.float32),
            pltpu.VMEM((_CHUNK, _D), jnp.float32),
            pltpu.VMEM((_RPW,), jnp.float32),
            pltpu.SemaphoreType.DMA,
            pltpu.SemaphoreType.DMA,
            pltpu.SemaphoreType.DMA,
            pltpu.SemaphoreType.DMA,
            pltpu.SemaphoreType.DMA,
        ],
    )(flat, starts, ends)


def kernel(flat, cu_seqlens):
    starts = cu_seqlens[:-1]
    ends = cu_seqlens[1:]
    return _padded_gather(flat, starts, ends)


# speedup vs baseline: 1.1221x; 1.1221x over previous
import functools

import jax
import jax.numpy as jnp
from jax import lax
from jax.experimental import pallas as pl
from jax.experimental.pallas import tpu as pltpu
from jax.experimental.pallas import tpu_sc as plsc

_B = 16
_L = 4096
_D = 512
_TOTAL = _B * _L // 2
_NW = 32
_RPW = _B * _L // _NW
_CHUNK = 64
_NCHUNK = _RPW // _CHUNK
_ZROWS = 32
_LANES = 16


def _sc_body(flat_hbm, starts_hbm, ends_hbm, out_hbm, mask_hbm,
             st_v, en_v, idx0, idx1, idx2, buf0, buf1, buf2, zbuf, mbuf,
             isem0, isem1, isem2, osem0, osem1, osem2, zsem):
    cid = lax.axis_index("c")
    sid = lax.axis_index("s")
    w = sid * 2 + cid
    b = w // 2
    t0 = (w % 2) * _RPW
    obase = w * _RPW

    pltpu.sync_copy(starts_hbm, st_v.at[pl.ds(0, _LANES)])
    pltpu.sync_copy(ends_hbm, en_v.at[pl.ds(0, _LANES)])
    iota = lax.iota(jnp.int32, _LANES)
    start_b = st_v[pl.ds(b, _LANES)][0]
    end_b = en_v[pl.ds(b, _LANES)][0]
    nv = jnp.clip(end_b - start_b - t0, 0, _RPW)
    s0 = start_b + t0
    pcv = (nv + _CHUNK - 1) // _CHUNK

    bufs = (buf0, buf1, buf2)
    idxs = (idx0, idx1, idx2)
    isems = (isem0, isem1, isem2)
    osems = (osem0, osem1, osem2)

    def start_in(c, bi):
        s = s0 + c * _CHUNK
        for kk in range(_CHUNK // _LANES):
            v = jnp.minimum(s + kk * _LANES + iota, _TOTAL - 1)
            idxs[bi][pl.ds(kk * _LANES, _LANES)] = v
        pltpu.make_async_copy(flat_hbm.at[idxs[bi]], bufs[bi], isems[bi]).start()

    for bi in range(3):
        @pl.when(bi < pcv)
        def _(bi=bi):
            start_in(bi, bi)

    zerosf = jnp.zeros((_LANES,), jnp.float32)

    def _zrow(row, _):
        for kk in range(_D // _LANES):
            zbuf[row, pl.ds(kk * _LANES, _LANES)] = zerosf
        return 0
    lax.fori_loop(0, _ZROWS, _zrow, 0)

    def _zfill(h, _):
        dst = pl.multiple_of(obase + pcv * _CHUNK + h * _ZROWS, _ZROWS)
        pltpu.make_async_copy(zbuf, out_hbm.at[pl.ds(dst, _ZROWS)], zsem).start()
        return 0
    nzfires = (_NCHUNK - pcv) * (_CHUNK // _ZROWS)
    lax.fori_loop(0, nzfires, _zfill, 0)

    for j in range(_RPW // _LANES):
        m = jnp.where(j * _LANES + iota < nv, 1.0, 0.0).astype(jnp.float32)
        mbuf[pl.ds(j * _LANES, _LANES)] = m
    pltpu.sync_copy(mbuf, mask_hbm.at[pl.ds(pl.multiple_of(obase, _RPW), _RPW)])

    def _pipe(g, _):
        for bi in range(3):
            c = g * 3 + bi

            @pl.when(c < pcv)
            def _(c=c, bi=bi):
                pltpu.make_async_copy(
                    flat_hbm.at[idxs[bi]], bufs[bi], isems[bi]
                ).wait()
                nvc = jnp.clip(nv - c * _CHUNK, 0, _CHUNK)

                def _ztail(row, _c):
                    for kk in range(_D // _LANES):
                        bufs[bi][row, pl.ds(kk * _LANES, _LANES)] = zerosf
                    return 0
                lax.fori_loop(nvc, _CHUNK, _ztail, 0)

                pltpu.make_async_copy(
                    bufs[bi],
                    out_hbm.at[pl.ds(
                        pl.multiple_of(obase + c * _CHUNK, _CHUNK), _CHUNK)],
                    osems[bi],
                ).start()

                @pl.when(c + 3 < pcv)
                def _():
                    pltpu.make_async_copy(
                        bufs[bi],
                        out_hbm.at[pl.ds(0, _CHUNK)],
                        osems[bi],
                    ).wait()
                    start_in(c + 3, bi)
        return 0
    lax.fori_loop(0, (pcv + 2) // 3, _pipe, 0)

    for bi in range(3):
        @pl.when(bi < pcv)
        def _(bi=bi):
            pltpu.make_async_copy(
                bufs[bi],
                out_hbm.at[pl.ds(0, _CHUNK)],
                osems[bi],
            ).wait()

    def _zdrain(h, _):
        pltpu.make_async_copy(zbuf, out_hbm.at[pl.ds(0, _ZROWS)], zsem).wait()
        return 0
    lax.fori_loop(0, nzfires, _zdrain, 0)


@jax.jit
def _padded_gather(flat, starts, ends):
    mesh = plsc.VectorSubcoreMesh(core_axis_name="c", subcore_axis_name="s")
    return pl.kernel(
        _sc_body,
        out_type=(
            jax.ShapeDtypeStruct((_B * _L, _D), jnp.float32),
            jax.ShapeDtypeStruct((_B * _L,), jnp.float32),
        ),
        mesh=mesh,
        scratch_types=[
            pltpu.VMEM((2 * _LANES,), jnp.int32),
            pltpu.VMEM((2 * _LANES,), jnp.int32),
            pltpu.VMEM((_CHUNK,), jnp.int32),
            pltpu.VMEM((_CHUNK,), jnp.int32),
            pltpu.VMEM((_CHUNK,), jnp.int32),
            pltpu.VMEM((_CHUNK, _D), jnp.float32),
            pltpu.VMEM((_CHUNK, _D), jnp.float32),
            pltpu.VMEM((_CHUNK, _D), jnp.float32),
            pltpu.VMEM((_ZROWS, _D), jnp.float32),
            pltpu.VMEM((_RPW,), jnp.float32),
            pltpu.SemaphoreType.DMA,
            pltpu.SemaphoreType.DMA,
            pltpu.SemaphoreType.DMA,
            pltpu.SemaphoreType.DMA,
            pltpu.SemaphoreType.DMA,
            pltpu.SemaphoreType.DMA,
            pltpu.SemaphoreType.DMA,
        ],
    )(flat, starts, ends)


def kernel(flat, cu_seqlens):
    starts = cu_seqlens[:-1]
    ends = cu_seqlens[1:]
    return _padded_gather(flat, starts, ends)


# speedup vs baseline: 1.1224x; 1.0002x over previous
import functools

import jax
import jax.numpy as jnp
from jax import lax
from jax.experimental import pallas as pl
from jax.experimental.pallas import tpu as pltpu
from jax.experimental.pallas import tpu_sc as plsc

_B = 16
_L = 4096
_D = 512
_TOTAL = _B * _L // 2
_NW = 32
_RPW = _B * _L // _NW
_CHUNK = 64
_NCHUNK = _RPW // _CHUNK
_ZROWS = 32
_LANES = 16


def _sc_body(flat_hbm, cu_hbm, out_hbm, mask_hbm,
             cu_v, idx0, idx1, idx2, buf0, buf1, buf2, zbuf, mbuf,
             isem0, isem1, isem2, osem0, osem1, osem2, zsem, msem):
    cid = lax.axis_index("c")
    sid = lax.axis_index("s")
    w = sid * 2 + cid
    b = w // 2
    t0 = (w % 2) * _RPW
    obase = w * _RPW

    pltpu.sync_copy(cu_hbm, cu_v.at[pl.ds(0, _B + 1)])
    iota = lax.iota(jnp.int32, _LANES)
    start_b = cu_v[pl.ds(b, _LANES)][0]
    end_b = cu_v[pl.ds(b + 1, _LANES)][0]
    nv = jnp.clip(end_b - start_b - t0, 0, _RPW)
    s0 = start_b + t0
    pcv = (nv + _CHUNK - 1) // _CHUNK

    bufs = (buf0, buf1, buf2)
    idxs = (idx0, idx1, idx2)
    isems = (isem0, isem1, isem2)
    osems = (osem0, osem1, osem2)

    def start_in(c, bi):
        s = s0 + c * _CHUNK
        for kk in range(_CHUNK // _LANES):
            v = jnp.minimum(s + kk * _LANES + iota, _TOTAL - 1)
            idxs[bi][pl.ds(kk * _LANES, _LANES)] = v
        pltpu.make_async_copy(flat_hbm.at[idxs[bi]], bufs[bi], isems[bi]).start()

    for bi in range(3):
        @pl.when(bi < pcv)
        def _(bi=bi):
            start_in(bi, bi)

    zerosf = jnp.zeros((_LANES,), jnp.float32)

    def _zrow(row, _):
        for kk in range(_D // _LANES):
            zbuf[row, pl.ds(kk * _LANES, _LANES)] = zerosf
        return 0
    lax.fori_loop(0, _ZROWS, _zrow, 0)

    def _zfill(h, _):
        dst = pl.multiple_of(obase + pcv * _CHUNK + h * _ZROWS, _ZROWS)
        pltpu.make_async_copy(zbuf, out_hbm.at[pl.ds(dst, _ZROWS)], zsem).start()
        return 0
    nzfires = (_NCHUNK - pcv) * (_CHUNK // _ZROWS)
    lax.fori_loop(0, nzfires, _zfill, 0)

    for j in range(_RPW // _LANES):
        m = jnp.where(j * _LANES + iota < nv, 1.0, 0.0).astype(jnp.float32)
        mbuf[pl.ds(j * _LANES, _LANES)] = m
    pltpu.make_async_copy(
        mbuf, mask_hbm.at[pl.ds(pl.multiple_of(obase, _RPW), _RPW)], msem
    ).start()

    def _pipe(g, _):
        for bi in range(3):
            c = g * 3 + bi

            @pl.when(c < pcv)
            def _(c=c, bi=bi):
                pltpu.make_async_copy(
                    flat_hbm.at[idxs[bi]], bufs[bi], isems[bi]
                ).wait()
                nvc = jnp.clip(nv - c * _CHUNK, 0, _CHUNK)

                def _ztail(row, _c):
                    for kk in range(_D // _LANES):
                        bufs[bi][row, pl.ds(kk * _LANES, _LANES)] = zerosf
                    return 0
                lax.fori_loop(nvc, _CHUNK, _ztail, 0)

                pltpu.make_async_copy(
                    bufs[bi],
                    out_hbm.at[pl.ds(
                        pl.multiple_of(obase + c * _CHUNK, _CHUNK), _CHUNK)],
                    osems[bi],
                ).start()

                @pl.when(c + 3 < pcv)
                def _():
                    pltpu.make_async_copy(
                        bufs[bi],
                        out_hbm.at[pl.ds(0, _CHUNK)],
                        osems[bi],
                    ).wait()
                    start_in(c + 3, bi)
        return 0
    lax.fori_loop(0, (pcv + 2) // 3, _pipe, 0)

    for bi in range(3):
        @pl.when(bi < pcv)
        def _(bi=bi):
            pltpu.make_async_copy(
                bufs[bi],
                out_hbm.at[pl.ds(0, _CHUNK)],
                osems[bi],
            ).wait()

    pltpu.make_async_copy(
        mbuf, mask_hbm.at[pl.ds(0, _RPW)], msem).wait()

    def _zdrain(h, _):
        pltpu.make_async_copy(zbuf, out_hbm.at[pl.ds(0, _ZROWS)], zsem).wait()
        return 0
    lax.fori_loop(0, nzfires, _zdrain, 0)


@jax.jit
def _padded_gather(flat, cu):
    mesh = plsc.VectorSubcoreMesh(core_axis_name="c", subcore_axis_name="s")
    return pl.kernel(
        _sc_body,
        out_type=(
            jax.ShapeDtypeStruct((_B * _L, _D), jnp.float32),
            jax.ShapeDtypeStruct((_B * _L,), jnp.float32),
        ),
        mesh=mesh,
        scratch_types=[
            pltpu.VMEM((3 * _LANES,), jnp.int32),
            pltpu.VMEM((_CHUNK,), jnp.int32),
            pltpu.VMEM((_CHUNK,), jnp.int32),
            pltpu.VMEM((_CHUNK,), jnp.int32),
            pltpu.VMEM((_CHUNK, _D), jnp.float32),
            pltpu.VMEM((_CHUNK, _D), jnp.float32),
            pltpu.VMEM((_CHUNK, _D), jnp.float32),
            pltpu.VMEM((_ZROWS, _D), jnp.float32),
            pltpu.VMEM((_RPW,), jnp.float32),
            pltpu.SemaphoreType.DMA,
            pltpu.SemaphoreType.DMA,
            pltpu.SemaphoreType.DMA,
            pltpu.SemaphoreType.DMA,
            pltpu.SemaphoreType.DMA,
            pltpu.SemaphoreType.DMA,
            pltpu.SemaphoreType.DMA,
            pltpu.SemaphoreType.DMA,
        ],
    )(flat, cu)


def kernel(flat, cu_seqlens):
    return _padded_gather(flat, cu_seqlens)
